# baseline (speedup 1.0000x reference)
import functools

import jax
import jax.numpy as jnp
from jax.experimental import pallas as pl

HEADS = 4
OPH = 32
HID = 128
NODE_BLK = 2000
EDGE_BLK = 10000


def _dense_body(xin_ref, bprev_ref, W_ref, As_ref, Ad_ref,
                h_ref, asr_ref, adt_ref, *, relu_in):
    x = xin_ref[...]
    if relu_in:
        x = jnp.maximum(x + bprev_ref[...], 0.0)
    if W_ref.shape[0] == 1:
        h = x * W_ref[...]
    else:
        h = jnp.dot(x, W_ref[...], preferred_element_type=jnp.float32)
    h_ref[...] = h
    asr_ref[...] = jnp.dot(h, As_ref[...], preferred_element_type=jnp.float32)
    adt_ref[...] = jnp.dot(h, Ad_ref[...], preferred_element_type=jnp.float32)


def _escore_body(es_ref, ed_ref, e_ref):
    s = es_ref[...] + ed_ref[...]
    e_ref[...] = jnp.where(s >= 0, s, 0.2 * s)


def _exp_body(e_ref, m_ref, ex_ref):
    ex_ref[...] = jnp.exp(e_ref[...] - m_ref[...])


def _msg_body(hs_ref, ex_ref, den_ref, R_ref, o_ref):
    alpha = ex_ref[...] / (den_ref[...] + 1e-16)
    af = jnp.dot(alpha, R_ref[...], preferred_element_type=jnp.float32)
    o_ref[...] = hs_ref[...] * af


def _clf_body(agg_ref, b3_ref, Wc1_ref, bc1_ref, Wc2_ref, bc2_ref, o_ref):
    t = jnp.maximum(agg_ref[...] + b3_ref[...], 0.0)
    t = jnp.dot(t, Wc1_ref[...], preferred_element_type=jnp.float32) + bc1_ref[...]
    t = jnp.maximum(t, 0.0)
    lg = jnp.dot(t, Wc2_ref[...], preferred_element_type=jnp.float32) + bc2_ref[...]
    m = jnp.max(lg, axis=-1, keepdims=True)
    o_ref[...] = lg - m - jnp.log(jnp.sum(jnp.exp(lg - m), axis=-1, keepdims=True))


def _rows(blk, cols):
    return pl.BlockSpec((blk, cols), lambda i: (i, 0))


def _full(r, c):
    return pl.BlockSpec((r, c), lambda i: (0, 0))


def _dense_call(xin, bprev, W, As, Ad, relu_in):
    n, d = xin.shape
    grid = (pl.cdiv(n, NODE_BLK),)
    return pl.pallas_call(
        functools.partial(_dense_body, relu_in=relu_in),
        grid=grid,
        in_specs=[_rows(NODE_BLK, d), _full(1, HID), _full(d, HID),
                  _full(HID, HEADS), _full(HID, HEADS)],
        out_specs=[_rows(NODE_BLK, HID), _rows(NODE_BLK, HEADS),
                   _rows(NODE_BLK, HEADS)],
        out_shape=[jax.ShapeDtypeStruct((n, HID), jnp.float32),
                   jax.ShapeDtypeStruct((n, HEADS), jnp.float32),
                   jax.ShapeDtypeStruct((n, HEADS), jnp.float32)],
    )(xin, bprev, W, As, Ad)


def _escore_call(es, ed):
    ne = es.shape[0]
    grid = (pl.cdiv(ne, EDGE_BLK),)
    return pl.pallas_call(
        _escore_body,
        grid=grid,
        in_specs=[_rows(EDGE_BLK, HEADS), _rows(EDGE_BLK, HEADS)],
        out_specs=_rows(EDGE_BLK, HEADS),
        out_shape=jax.ShapeDtypeStruct((ne, HEADS), jnp.float32),
    )(es, ed)


def _exp_call(e, mg):
    ne = e.shape[0]
    grid = (pl.cdiv(ne, EDGE_BLK),)
    return pl.pallas_call(
        _exp_body,
        grid=grid,
        in_specs=[_rows(EDGE_BLK, HEADS), _rows(EDGE_BLK, HEADS)],
        out_specs=_rows(EDGE_BLK, HEADS),
        out_shape=jax.ShapeDtypeStruct((ne, HEADS), jnp.float32),
    )(e, mg)


def _msg_call(hs, ex, deng, R):
    ne = hs.shape[0]
    grid = (pl.cdiv(ne, EDGE_BLK),)
    return pl.pallas_call(
        _msg_body,
        grid=grid,
        in_specs=[_rows(EDGE_BLK, HID), _rows(EDGE_BLK, HEADS),
                  _rows(EDGE_BLK, HEADS), _full(HEADS, HID)],
        out_specs=_rows(EDGE_BLK, HID),
        out_shape=jax.ShapeDtypeStruct((ne, HID), jnp.float32),
    )(hs, ex, deng, R)


def _clf_call(agg, b3, Wc1, bc1, Wc2, bc2):
    n = agg.shape[0]
    nc = Wc2.shape[1]
    grid = (pl.cdiv(n, NODE_BLK),)
    return pl.pallas_call(
        _clf_body,
        grid=grid,
        in_specs=[_rows(NODE_BLK, HID), _full(1, HID),
                  _full(HID, HID // 2), _full(1, HID // 2),
                  _full(HID // 2, nc), _full(1, nc)],
        out_specs=_rows(NODE_BLK, nc),
        out_shape=jax.ShapeDtypeStruct((n, nc), jnp.float32),
    )(agg, b3, Wc1, bc1, Wc2, bc2)


def kernel(x, edge_index, W1, a_s1, a_d1, b1, W2, a_s2, a_d2, b2,
           W3, a_s3, a_d3, b3, Wc1, bc1, Wc2, bc2):
    n = x.shape[0]
    src = edge_index[0]
    dst = edge_index[1]
    loop = jnp.arange(n, dtype=src.dtype)
    src = jnp.concatenate([src, loop])
    dst = jnp.concatenate([dst, loop])

    R = jnp.repeat(jnp.eye(HEADS, dtype=jnp.float32), OPH, axis=1)
    RT = R.T

    def head_mat(a):
        return RT * a.reshape(-1)[:, None]

    zeros_b = jnp.zeros((1, HID), jnp.float32)

    def gat_layer(xin, bprev, relu_in, W, a_s, a_d):
        h, asr, adt = _dense_call(xin, bprev, W, head_mat(a_s),
                                  head_mat(a_d), relu_in)
        es = jnp.take(asr, src, axis=0)
        ed = jnp.take(adt, dst, axis=0)
        e = _escore_call(es, ed)
        emax = jax.ops.segment_max(e, dst, num_segments=n)
        ex = _exp_call(e, jnp.take(emax, dst, axis=0))
        den = jax.ops.segment_sum(ex, dst, num_segments=n)
        hs = jnp.take(h, src, axis=0)
        msg = _msg_call(hs, ex, jnp.take(den, dst, axis=0), R)
        return jax.ops.segment_sum(msg, dst, num_segments=n)

    agg1 = gat_layer(x, zeros_b, False, W1, a_s1, a_d1)
    agg2 = gat_layer(agg1, b1.reshape(1, HID), True, W2, a_s2, a_d2)
    agg3 = gat_layer(agg2, b2.reshape(1, HID), True, W3, a_s3, a_d3)
    return _clf_call(agg3, b3.reshape(1, HID), Wc1, bc1.reshape(1, -1),
                     Wc2, bc2.reshape(1, -1))
